# Initial kernel scaffold; baseline (speedup 1.0000x reference)
import jax
import jax.numpy as jnp
from jax.experimental import pallas as pl


def kernel(x, edge_index, W1, b1, W2, b2):
    raise NotImplementedError("write your pallas kernel here")


# speedup vs baseline: 26.0477x; 26.0477x over previous
import functools

import jax
import jax.numpy as jnp
from jax import lax
from jax.experimental import pallas as pl
from jax.experimental.pallas import tpu as pltpu
from jax.experimental.pallas import tpu_sc as plsc

N = 10000
E = 320000
D = 128
H = 128
C = 7
CP = 16

NC = 2
NS = 16
NW = NC * NS

NPAD = 10240
RPT = NPAD // NS

CHUNK = 128
NCHT = 80
EPT = NCHT * CHUNK
EPAD = NW * EPT
NCH2 = EPAD // CHUNK
WCH = 16
NWIN = NCHT // WCH
NBUF = 2


@functools.cache
def _mesh():
    return plsc.VectorSubcoreMesh(core_axis_name="c", subcore_axis_name="s",
                                  num_cores=NC, num_subcores=NS)


def _hist_body(dst2, out, acc, zbuf, ones_b, dst_all):
    c = lax.axis_index("c")
    s = lax.axis_index("s")
    w = s * NC + c
    zv = jnp.zeros((16,), jnp.float32)
    ov = jnp.ones((16,), jnp.float32)

    def _zero(i, carry):
        zbuf[pl.ds(i * 16, 16)] = zv
        return carry
    lax.fori_loop(0, RPT // 16, _zero, 0)
    for k in range(CHUNK // 16):
        ones_b[pl.ds(k * 16, 16)] = ov
    pltpu.sync_copy(zbuf, acc.at[pl.ds(s * RPT, RPT)])
    pltpu.sync_copy(dst2.at[pl.ds(w * NCHT, NCHT)], dst_all)
    plsc.subcore_barrier()

    def _chunk(j, carry):
        pltpu.sync_copy(ones_b, acc.at[dst_all.at[j]], add=True)
        return carry
    lax.fori_loop(0, NCHT, _chunk, 0)
    plsc.subcore_barrier()
    pltpu.sync_copy(acc.at[pl.ds(s * RPT, RPT)],
                    out.at[c, pl.ds(s * RPT, RPT)])


@functools.cache
def _hist_call():
    return pl.kernel(
        _hist_body,
        out_type=jax.ShapeDtypeStruct((NC, NPAD), jnp.float32),
        mesh=_mesh(),
        scratch_types=[
            pltpu.VMEM_SHARED((NPAD,), jnp.float32),
            pltpu.VMEM((RPT,), jnp.float32),
            pltpu.VMEM((CHUNK,), jnp.float32),
            pltpu.VMEM((NCHT, CHUNK), jnp.int32),
        ],
    )


def _agg_body(y, src2, dst2, out, acc, zbuf, src_w, dst_w,
              rb0, rb1, sem0, sem1):
    c = lax.axis_index("c")
    s = lax.axis_index("s")
    w = s * NC + c
    zv = jnp.zeros((16,), jnp.float32)

    def _zero(r, carry):
        for k in range(H // 16):
            zbuf[r, pl.ds(k * 16, 16)] = zv
        return carry
    lax.fori_loop(0, 16, _zero, 0)
    for t in range(RPT // 16):
        pltpu.sync_copy(zbuf, acc.at[pl.ds(s * RPT + t * 16, 16)])
    plsc.subcore_barrier()

    base = w * NCHT
    rbs = (rb0, rb1)
    sems = (sem0, sem1)

    def _win(wi, carry):
        pltpu.sync_copy(src2.at[pl.ds(base + wi * WCH, WCH)], src_w)
        pltpu.sync_copy(dst2.at[pl.ds(base + wi * WCH, WCH)], dst_w)
        for b in range(NBUF):
            pltpu.async_copy(y.at[src_w.at[b]], rbs[b], sems[b])

        def _inner(g, carry2):
            for b in range(NBUF):
                j = g * NBUF + b
                pltpu.make_async_copy(y.at[src_w.at[j]], rbs[b],
                                      sems[b]).wait()
                pltpu.sync_copy(rbs[b], acc.at[dst_w.at[j]], add=True)

                @pl.when(j + NBUF < WCH)
                def _prefetch():
                    pltpu.async_copy(y.at[src_w.at[j + NBUF]], rbs[b],
                                     sems[b])
            return carry2
        lax.fori_loop(0, WCH // NBUF, _inner, 0)
        return carry
    lax.fori_loop(0, NWIN, _win, 0)
    plsc.subcore_barrier()
    pltpu.sync_copy(acc.at[pl.ds(s * RPT, RPT)],
                    out.at[c, pl.ds(s * RPT, RPT)])


@functools.cache
def _agg_call():
    return pl.kernel(
        _agg_body,
        out_type=jax.ShapeDtypeStruct((NC, NPAD, H), jnp.float32),
        mesh=_mesh(),
        scratch_types=[
            pltpu.VMEM_SHARED((NPAD, H), jnp.float32),
            pltpu.VMEM((16, H), jnp.float32),
            pltpu.VMEM((WCH, CHUNK), jnp.int32),
            pltpu.VMEM((WCH, CHUNK), jnp.int32),
            pltpu.VMEM((CHUNK, H), jnp.float32),
            pltpu.VMEM((CHUNK, H), jnp.float32),
            pltpu.SemaphoreType.DMA,
            pltpu.SemaphoreType.DMA,
        ],
    )


BLKR = 256


def _tc1_body(x_ref, w1_ref, hist_ref, y_ref, dinv_ref):
    deg = hist_ref[0] + hist_ref[1] + 1.0
    dinv = lax.rsqrt(deg)
    y = jnp.dot(x_ref[...], w1_ref[...], preferred_element_type=jnp.float32)
    y_ref[...] = y * dinv
    dinv_ref[...] = dinv


_tc1_call = pl.pallas_call(
    _tc1_body,
    grid=(NPAD // BLKR,),
    in_specs=[
        pl.BlockSpec((BLKR, D), lambda i: (i, 0)),
        pl.BlockSpec((D, H), lambda i: (0, 0)),
        pl.BlockSpec((NC, BLKR, 1), lambda i: (0, i, 0)),
    ],
    out_specs=[
        pl.BlockSpec((BLKR, H), lambda i: (i, 0)),
        pl.BlockSpec((BLKR, 1), lambda i: (i, 0)),
    ],
    out_shape=[
        jax.ShapeDtypeStruct((NPAD, H), jnp.float32),
        jax.ShapeDtypeStruct((NPAD, 1), jnp.float32),
    ],
)


def _tc2_body(s_ref, y1_ref, dinv_ref, b1_ref, z_ref):
    dinv = dinv_ref[...]
    agg = (s_ref[0] + s_ref[1] + y1_ref[...]) * dinv
    z_ref[...] = jnp.maximum(agg + b1_ref[...], 0.0) * dinv


_tc2_call = pl.pallas_call(
    _tc2_body,
    grid=(NPAD // BLKR,),
    in_specs=[
        pl.BlockSpec((NC, BLKR, H), lambda i: (0, i, 0)),
        pl.BlockSpec((BLKR, H), lambda i: (i, 0)),
        pl.BlockSpec((BLKR, 1), lambda i: (i, 0)),
        pl.BlockSpec((1, H), lambda i: (0, 0)),
    ],
    out_specs=pl.BlockSpec((BLKR, H), lambda i: (i, 0)),
    out_shape=jax.ShapeDtypeStruct((NPAD, H), jnp.float32),
)


def _tc3_body(s_ref, z_ref, dinv_ref, w2_ref, b2_ref, o_ref):
    agg = (s_ref[0] + s_ref[1] + z_ref[...]) * dinv_ref[...]
    o_ref[...] = (jnp.dot(agg, w2_ref[...],
                          preferred_element_type=jnp.float32)
                  + b2_ref[...])


_tc3_call = pl.pallas_call(
    _tc3_body,
    grid=(NPAD // BLKR,),
    in_specs=[
        pl.BlockSpec((NC, BLKR, H), lambda i: (0, i, 0)),
        pl.BlockSpec((BLKR, H), lambda i: (i, 0)),
        pl.BlockSpec((BLKR, 1), lambda i: (i, 0)),
        pl.BlockSpec((H, CP), lambda i: (0, 0)),
        pl.BlockSpec((1, CP), lambda i: (0, 0)),
    ],
    out_specs=pl.BlockSpec((BLKR, CP), lambda i: (i, 0)),
    out_shape=jax.ShapeDtypeStruct((NPAD, CP), jnp.float32),
)


def kernel(x, edge_index, W1, b1, W2, b2):
    src = edge_index[0]
    dst = edge_index[1]
    pad_idx = (N + (jnp.arange(EPAD - E, dtype=jnp.int32) % (NPAD - N)))
    src_p = jnp.concatenate([src, pad_idx]).reshape(NCH2, CHUNK)
    dst_p = jnp.concatenate([dst, pad_idx]).reshape(NCH2, CHUNK)
    x_pad = jnp.pad(x, ((0, NPAD - N), (0, 0)))
    w2_p = jnp.pad(W2, ((0, 0), (0, CP - C)))
    b1_r = b1.reshape(1, H)
    b2_r = jnp.pad(b2, (0, CP - C)).reshape(1, CP)

    hist = _hist_call()(dst_p).reshape(NC, NPAD, 1)
    y1, dinv = _tc1_call(x_pad, W1, hist)
    s1 = _agg_call()(y1, src_p, dst_p)
    z = _tc2_call(s1, y1, dinv, b1_r)
    s2 = _agg_call()(z, src_p, dst_p)
    outp = _tc3_call(s2, z, dinv, w2_p, b2_r)
    return outp[:N, :C]
